# baseline (speedup 1.0000x reference)
import functools

import jax
import jax.numpy as jnp
from jax import lax
from jax.experimental import pallas as pl
from jax.experimental.pallas import tpu as pltpu

KK = 1024
DD = 64
BETA = 0.25


def _vq_body(ze_ref, e_ref, et_ref, zq_ref, idx_ref, sse_ref):
    zb = ze_ref[0].reshape(DD, ze_ref.shape[2] * ze_ref.shape[3])
    e = e_ref[...]
    et = et_ref[...]
    z2 = jnp.sum(zb * zb, axis=0, keepdims=True)
    e2 = jnp.sum(e * e, axis=1, keepdims=True)
    mm = lax.dot_general(e, zb, (((1,), (0,)), ((), ())),
                         preferred_element_type=jnp.float32)
    d = (z2 + e2) - 2.0 * mm
    m = jnp.min(d, axis=0, keepdims=True)
    iota = lax.broadcasted_iota(jnp.int32, d.shape, 0)
    idx = jnp.min(jnp.where(d == m, iota, KK), axis=0, keepdims=True)
    onehot = jnp.where(iota == idx, 1.0, 0.0)
    zq = lax.dot_general(et, onehot, (((1,), (0,)), ((), ())),
                         preferred_element_type=jnp.float32)
    out = zb + (zq - zb)
    zq_ref[0] = out.reshape(DD, ze_ref.shape[2], ze_ref.shape[3])
    idx_ref[0] = idx

    @pl.when(pl.program_id(0) == 0)
    def _():
        sse_ref[0, 0] = 0.0

    sse_ref[0, 0] += jnp.sum(m)


def _vq(ze, e):
    b, _, h, w = ze.shape
    hw = h * w
    return pl.pallas_call(
        _vq_body,
        grid=(b,),
        in_specs=[
            pl.BlockSpec((1, DD, h, w), lambda i: (i, 0, 0, 0)),
            pl.BlockSpec((KK, DD), lambda i: (0, 0)),
            pl.BlockSpec((DD, KK), lambda i: (0, 0)),
        ],
        out_specs=[
            pl.BlockSpec((1, DD, h, w), lambda i: (i, 0, 0, 0)),
            pl.BlockSpec((1, 1, hw), lambda i: (i, 0, 0)),
            pl.BlockSpec((1, 1), lambda i: (0, 0), memory_space=pltpu.SMEM),
        ],
        out_shape=[
            jax.ShapeDtypeStruct((b, DD, h, w), jnp.float32),
            jax.ShapeDtypeStruct((b, 1, hw), jnp.int32),
            jax.ShapeDtypeStruct((1, 1), jnp.float32),
        ],
    )(ze, e, e.T)


def kernel(z_e, codebook):
    b, c, h, w = z_e.shape
    hw = h * w
    zq, idx3, sse = _vq(z_e, codebook)
    commit = BETA * (sse[0, 0] / jnp.float32(b * c * hw))
    indices_out = idx3.reshape(b, h, w)
    codebook_loss = jnp.zeros(())
    return (zq, codebook_loss, commit, commit, indices_out)


# speedup vs baseline: 1.4745x; 1.4745x over previous
import functools

import jax
import jax.numpy as jnp
from jax import lax
from jax.experimental import pallas as pl
from jax.experimental.pallas import tpu as pltpu

KK = 1024
DD = 64
BETA = 0.25
FBIAS = 0x3F800000


def _vq_body(ze_ref, e_ref, et_ref, biota_ref, zq_ref, idx_ref, sse_ref):
    zb = ze_ref[0]
    e = e_ref[...]
    et = et_ref[...]
    biota = biota_ref[...]
    z2 = jnp.sum(zb * zb, axis=0, keepdims=True)
    e2 = jnp.sum(e * e, axis=1, keepdims=True)
    mm2 = lax.dot_general(e + e, zb, (((1,), (0,)), ((), ())),
                          preferred_element_type=jnp.float32)
    d = (z2 + e2) - mm2
    m = jnp.min(d, axis=0, keepdims=True)
    idxf = jnp.min(jnp.where(d == m, biota, 2.0), axis=0, keepdims=True)
    onehot = jnp.where(biota == idxf, 1.0, 0.0)
    zq = lax.dot_general(et, onehot, (((1,), (0,)), ((), ())),
                         preferred_element_type=jnp.float32)
    zq_ref[0] = zb + (zq - zb)
    idx_ref[0] = lax.bitcast_convert_type(idxf, jnp.int32) - FBIAS

    @pl.when(pl.program_id(0) == 0)
    def _():
        sse_ref[0, 0] = 0.0

    sse_ref[0, 0] += jnp.sum(m)


def _vq(ze3, e):
    b = ze3.shape[0]
    hw = ze3.shape[2]
    biota = lax.bitcast_convert_type(
        jnp.broadcast_to(
            (jnp.arange(KK, dtype=jnp.int32) | FBIAS)[:, None], (KK, hw)
        ),
        jnp.float32,
    )
    return pl.pallas_call(
        _vq_body,
        grid=(b,),
        in_specs=[
            pl.BlockSpec((1, DD, hw), lambda i: (i, 0, 0)),
            pl.BlockSpec((KK, DD), lambda i: (0, 0)),
            pl.BlockSpec((DD, KK), lambda i: (0, 0)),
            pl.BlockSpec((KK, hw), lambda i: (0, 0)),
        ],
        out_specs=[
            pl.BlockSpec((1, DD, hw), lambda i: (i, 0, 0)),
            pl.BlockSpec((1, 1, hw), lambda i: (i, 0, 0)),
            pl.BlockSpec((1, 1), lambda i: (0, 0), memory_space=pltpu.SMEM),
        ],
        out_shape=[
            jax.ShapeDtypeStruct((b, DD, hw), jnp.float32),
            jax.ShapeDtypeStruct((b, 1, hw), jnp.int32),
            jax.ShapeDtypeStruct((1, 1), jnp.float32),
        ],
    )(ze3, e, e.T, biota)


def kernel(z_e, codebook):
    b, c, h, w = z_e.shape
    hw = h * w
    ze3 = z_e.reshape(b, c, hw)
    zq3, idx3, sse = _vq(ze3, codebook)
    commit = BETA * (sse[0, 0] / jnp.float32(b * c * hw))
    z_q_out = zq3.reshape(b, c, h, w)
    indices_out = idx3.reshape(b, h, w)
    codebook_loss = jnp.zeros(())
    return (z_q_out, codebook_loss, commit, commit, indices_out)
